# baseline (speedup 1.0000x reference)
import jax
import jax.numpy as jnp
from jax import lax
from jax.experimental import pallas as pl
from jax.experimental.pallas import tpu as pltpu
from jax.experimental.pallas import tpu_sc as plsc

F32 = jnp.float32
I32 = jnp.int32
NC = 2
NS = 16
GROUP = 128
LANES = 128
HIGHEST = lax.Precision.HIGHEST


def _round_up(a, b):
    return (a + b - 1) // b * b


def _sc_mesh():
    return plsc.VectorSubcoreMesh(
        core_axis_name="c", subcore_axis_name="s",
        num_cores=NC, num_subcores=NS)


def _make_deg_kernel(n, ngd, npad):
    nw = NC * NS

    def body(dst_hbm, out_hbm, deg_v, dst_v):
        cid = lax.axis_index("c")
        sid = lax.axis_index("s")
        wid = cid * NS + sid
        pltpu.sync_copy(dst_hbm.at[wid], dst_v)

        def zstep(i, c):
            deg_v[pl.ds(i * 16, 16)] = jnp.zeros((16,), F32)
            return c
        lax.fori_loop(0, npad // 16, zstep, 0)

        ones = jnp.ones((16,), F32)

        def estep(g, c):
            for j in range(GROUP // 16):
                idx = dst_v[pl.ds(g * GROUP + j * 16, 16)]
                plsc.addupdate_scatter(deg_v, [idx], ones)
            return c
        lax.fori_loop(0, ngd, estep, 0)
        pltpu.sync_copy(deg_v, out_hbm.at[wid])

    return pl.kernel(
        body,
        out_type=jax.ShapeDtypeStruct((nw, npad), F32),
        mesh=_sc_mesh(),
        compiler_params=pltpu.CompilerParams(needs_layout_passes=False),
        scratch_types=[
            pltpu.VMEM((npad,), F32),
            pltpu.VMEM((ngd * GROUP,), I32),
        ],
    )


def _make_agg_kernel(n, ng, nchunks):
    npad = n + 8
    rpt = _round_up(-(-n // NS), 8)
    last = n - rpt * (NS - 1)
    assert last > 0
    half = nchunks // 2

    def body(*refs):
        vs = refs[:nchunks]
        src_hbm = refs[nchunks]
        dst_hbm = refs[nchunks + 1]
        gs = refs[nchunks + 2:2 * nchunks + 2]
        accum, src_v, dst_v, rows, sem = refs[2 * nchunks + 2:]
        cid = lax.axis_index("c")
        sid = lax.axis_index("s")
        pltpu.sync_copy(src_hbm.at[sid], src_v)
        pltpu.sync_copy(dst_hbm.at[sid], dst_v)

        def one_pass(v_ref, g_ref):
            off = pl.multiple_of(sid * rpt, 8)

            @pl.when(sid < NS - 1)
            def _():
                pltpu.sync_copy(v_ref.at[pl.ds(off, rpt)],
                                accum.at[pl.ds(off, rpt)])

            @pl.when(sid == NS - 1)
            def _():
                pltpu.sync_copy(v_ref.at[pl.ds((NS - 1) * rpt, last)],
                                accum.at[pl.ds((NS - 1) * rpt, last)])

            plsc.subcore_barrier()

            def step(g, c):
                pltpu.async_copy(v_ref.at[src_v.at[g]], rows, sem).wait()
                pltpu.sync_copy(rows, accum.at[dst_v.at[g]], add=True)
                return c
            lax.fori_loop(0, ng, step, 0)

            plsc.subcore_barrier()

            @pl.when(sid < NS - 1)
            def _():
                pltpu.sync_copy(accum.at[pl.ds(off, rpt)],
                                g_ref.at[pl.ds(off, rpt)])

            @pl.when(sid == NS - 1)
            def _():
                pltpu.sync_copy(accum.at[pl.ds((NS - 1) * rpt, last)],
                                g_ref.at[pl.ds((NS - 1) * rpt, last)])

        for p in range(half):
            @pl.when(cid == 0)
            def _(p=p):
                one_pass(vs[p], gs[p])

            @pl.when(cid == 1)
            def _(p=p):
                one_pass(vs[p + half], gs[p + half])

    return pl.kernel(
        body,
        out_type=tuple(jax.ShapeDtypeStruct((n, LANES), F32)
                       for _ in range(nchunks)),
        mesh=_sc_mesh(),
        scratch_types=[
            pltpu.VMEM_SHARED((npad, LANES), F32),
            pltpu.VMEM((ng, GROUP), I32),
            pltpu.VMEM((ng, GROUP), I32),
            pltpu.VMEM((GROUP, LANES), F32),
            pltpu.SemaphoreType.DMA,
        ],
    )


def _prep(x, degp, n, bn):
    grid = -(-n // bn)
    nw, npad = degp.shape
    d_in = x.shape[1]

    def body(x_ref, dp_ref, srep_ref, x0_ref, x1_ref):
        d = jnp.sum(dp_ref[...], axis=0)[:, None] + 1.0
        s = lax.rsqrt(d)
        srep_ref[...] = jnp.broadcast_to(s, (bn, LANES))
        xb = x_ref[...]
        x0_ref[...] = xb[:, :LANES] * s
        x1_ref[...] = xb[:, LANES:] * s

    return pl.pallas_call(
        body,
        grid=(grid,),
        in_specs=[
            pl.BlockSpec((bn, d_in), lambda i: (i, 0)),
            pl.BlockSpec((nw, bn), lambda i: (0, i)),
        ],
        out_specs=[pl.BlockSpec((bn, LANES), lambda i: (i, 0))] * 3,
        out_shape=[jax.ShapeDtypeStruct((n, LANES), F32)] * 3,
    )(x, degp)


def _m1(g0, g1, srep, W1, b1, n, bn):
    grid = -(-n // bn)
    d_in, d_h = W1.shape

    def body(g0_ref, g1_ref, s_ref, w_ref, b_ref, *outs):
        scol = s_ref[...][:, 0:1]
        a = jnp.concatenate([g0_ref[...], g1_ref[...]], axis=1) * scol
        u = lax.dot_general(a, w_ref[...], (((1,), (0,)), ((), ())),
                            precision=HIGHEST, preferred_element_type=F32)
        h = jnp.maximum(u + b_ref[...], 0.0) * scol
        for c, o in enumerate(outs):
            o[...] = h[:, c * LANES:(c + 1) * LANES]

    nchunks = d_h // LANES
    return pl.pallas_call(
        body,
        grid=(grid,),
        in_specs=[
            pl.BlockSpec((bn, LANES), lambda i: (i, 0)),
            pl.BlockSpec((bn, LANES), lambda i: (i, 0)),
            pl.BlockSpec((bn, LANES), lambda i: (i, 0)),
            pl.BlockSpec((d_in, d_h), lambda i: (0, 0)),
            pl.BlockSpec((1, d_h), lambda i: (0, 0)),
        ],
        out_specs=[pl.BlockSpec((bn, LANES), lambda i: (i, 0))] * nchunks,
        out_shape=[jax.ShapeDtypeStruct((n, LANES), F32)] * nchunks,
    )(g0, g1, srep, W1, b1)


def _m23(gs, srep, W2, b2, W3, n, bn):
    grid = -(-n // bn)
    d_h = W2.shape[0]
    d_emb = W3.shape[1]
    nin = len(gs)

    def body(*refs):
        g_refs = refs[:nin]
        s_ref, w2_ref, b2_ref, w3_ref = refs[nin:nin + 4]
        outs = refs[nin + 4:]
        scol = s_ref[...][:, 0:1]
        a = jnp.concatenate([g[...] for g in g_refs], axis=1) * scol
        u = lax.dot_general(a, w2_ref[...], (((1,), (0,)), ((), ())),
                            precision=HIGHEST, preferred_element_type=F32)
        h = jnp.maximum(u + b2_ref[...], 0.0)
        t = lax.dot_general(h, w3_ref[...], (((1,), (0,)), ((), ())),
                            precision=HIGHEST, preferred_element_type=F32)
        t = t * scol
        for c, o in enumerate(outs):
            o[...] = t[:, c * LANES:(c + 1) * LANES]

    nchunks = d_emb // LANES
    return pl.pallas_call(
        body,
        grid=(grid,),
        in_specs=(
            [pl.BlockSpec((bn, LANES), lambda i: (i, 0))] * (nin + 1)
            + [
                pl.BlockSpec((d_h, d_h), lambda i: (0, 0)),
                pl.BlockSpec((1, d_h), lambda i: (0, 0)),
                pl.BlockSpec((d_h, d_emb), lambda i: (0, 0)),
            ]
        ),
        out_specs=[pl.BlockSpec((bn, LANES), lambda i: (i, 0))] * nchunks,
        out_shape=[jax.ShapeDtypeStruct((n, LANES), F32)] * nchunks,
    )(*gs, srep, W2, b2, W3)


def _pool(g0, g1, srep, b3, batch3, Wc, bc, n, bn, n_graphs):
    grid = -(-n // bn)
    d_emb = b3.shape[1]
    n_cls = Wc.shape[1]

    def body(g0_ref, g1_ref, s_ref, b3_ref, bt_ref, wc_ref, bc_ref,
             emb_ref, log_ref, sums, cnts):
        i = pl.program_id(0)

        @pl.when(i == 0)
        def _():
            sums[...] = jnp.zeros_like(sums)
            cnts[...] = jnp.zeros_like(cnts)

        scol = s_ref[...][:, 0:1]
        v = (jnp.concatenate([g0_ref[...], g1_ref[...]], axis=1) * scol
             + b3_ref[...])
        li = lax.broadcasted_iota(I32, (bn, 1), 0)
        valid = (i * bn + li) < n
        v = jnp.where(valid, v, 0.0)
        b = bt_ref[...][0, 0, :]
        gi = lax.broadcasted_iota(I32, (n_graphs, bn), 0)
        m = (gi == b[None, :]).astype(F32)
        sums[...] += lax.dot_general(m, v, (((1,), (0,)), ((), ())),
                                     precision=HIGHEST,
                                     preferred_element_type=F32)
        cnts[...] += jnp.broadcast_to(
            jnp.sum(m, axis=1, keepdims=True), cnts.shape)

        @pl.when(i == grid - 1)
        def _():
            cnt = jnp.maximum(cnts[...][:, 0:1], 1.0)
            emb = sums[...] / cnt
            emb_ref[...] = emb
            log_ref[...] = lax.dot_general(
                emb, wc_ref[...], (((1,), (0,)), ((), ())),
                precision=HIGHEST, preferred_element_type=F32) + bc_ref[...]

    return pl.pallas_call(
        body,
        grid=(grid,),
        in_specs=[
            pl.BlockSpec((bn, LANES), lambda i: (i, 0)),
            pl.BlockSpec((bn, LANES), lambda i: (i, 0)),
            pl.BlockSpec((bn, LANES), lambda i: (i, 0)),
            pl.BlockSpec((1, d_emb), lambda i: (0, 0)),
            pl.BlockSpec((1, 1, bn), lambda i: (i, 0, 0)),
            pl.BlockSpec((d_emb, n_cls), lambda i: (0, 0)),
            pl.BlockSpec((1, n_cls), lambda i: (0, 0)),
        ],
        out_specs=[
            pl.BlockSpec((n_graphs, d_emb), lambda i: (0, 0)),
            pl.BlockSpec((n_graphs, n_cls), lambda i: (0, 0)),
        ],
        out_shape=[
            jax.ShapeDtypeStruct((n_graphs, d_emb), F32),
            jax.ShapeDtypeStruct((n_graphs, n_cls), F32),
        ],
        scratch_shapes=[
            pltpu.VMEM((n_graphs, d_emb), F32),
            pltpu.VMEM((n_graphs, LANES), F32),
        ],
    )(g0, g1, srep, b3, batch3, Wc, bc)


def kernel(x, edge_index, batch, W1, b1, W2, b2, W3, b3, Wc, bc):
    n, d_in = x.shape
    e = edge_index.shape[1]
    n_graphs = 64
    bn = 1024
    src = edge_index[0].astype(I32)
    dst = edge_index[1].astype(I32)

    ept = _round_up(-(-e // NS), GROUP)
    ng = ept // GROUP
    pad16 = NS * ept - e
    skey = jnp.sort((src << 16) | dst)
    src16 = jnp.concatenate(
        [lax.shift_right_logical(skey, 16),
         jnp.full((pad16,), n - 1, I32)]).reshape(NS, ng, GROUP)
    dst16 = jnp.concatenate(
        [skey & 0xFFFF, jnp.full((pad16,), n, I32)]).reshape(NS, ng, GROUP)

    nw = NC * NS
    eptd = _round_up(-(-e // nw), GROUP)
    ngd = eptd // GROUP
    pad32 = nw * eptd - e
    dstd = jnp.concatenate(
        [dst, jnp.full((pad32,), n, I32)]).reshape(nw, eptd)

    npad_deg = _round_up(n + 1, bn)
    degp = _make_deg_kernel(n, ngd, npad_deg)(dstd)
    srep, x0, x1 = _prep(x, degp, n, bn)

    g10, g11 = _make_agg_kernel(n, ng, 2)(x0, x1, src16, dst16)
    hs = _m1(g10, g11, srep, W1, b1.reshape(1, -1), n, bn)
    g2 = _make_agg_kernel(n, ng, 4)(*hs, src16, dst16)
    t0, t1 = _m23(g2, srep, W2, b2.reshape(1, -1), W3, n, bn)
    g30, g31 = _make_agg_kernel(n, ng, 2)(t0, t1, src16, dst16)

    ngrid = -(-n // bn)
    batch_pad = jnp.concatenate(
        [batch.astype(I32), jnp.full((ngrid * bn - n,), n_graphs, I32)])
    batch3 = batch_pad.reshape(ngrid, 1, bn)
    emb, logits = _pool(g30, g31, srep, b3.reshape(1, -1), batch3,
                        Wc, bc.reshape(1, -1), n, bn, n_graphs)
    return emb, logits


# speedup vs baseline: 1.1014x; 1.1014x over previous
import jax
import jax.numpy as jnp
from jax import lax
from jax.experimental import pallas as pl
from jax.experimental.pallas import tpu as pltpu
from jax.experimental.pallas import tpu_sc as plsc

F32 = jnp.float32
I32 = jnp.int32
NC = 2
NS = 16
GROUP = 128
LANES = 128
HIGHEST = lax.Precision.HIGHEST


def _round_up(a, b):
    return (a + b - 1) // b * b


def _sc_mesh():
    return plsc.VectorSubcoreMesh(
        core_axis_name="c", subcore_axis_name="s",
        num_cores=NC, num_subcores=NS)


def _make_deg_kernel(n, ngd, npad):
    nw = NC * NS

    def body(dst_hbm, out_hbm, deg_v, dst_v):
        cid = lax.axis_index("c")
        sid = lax.axis_index("s")
        wid = cid * NS + sid
        pltpu.sync_copy(dst_hbm.at[wid], dst_v)

        def zstep(i, c):
            deg_v[pl.ds(i * 16, 16)] = jnp.zeros((16,), F32)
            return c
        lax.fori_loop(0, npad // 16, zstep, 0)

        ones = jnp.ones((16,), F32)

        def estep(g, c):
            for j in range(GROUP // 16):
                idx = dst_v[pl.ds(g * GROUP + j * 16, 16)]
                plsc.addupdate_scatter(deg_v, [idx], ones)
            return c
        lax.fori_loop(0, ngd, estep, 0)
        pltpu.sync_copy(deg_v, out_hbm.at[wid])

    return pl.kernel(
        body,
        out_type=jax.ShapeDtypeStruct((nw, npad), F32),
        mesh=_sc_mesh(),
        compiler_params=pltpu.CompilerParams(needs_layout_passes=False),
        scratch_types=[
            pltpu.VMEM((npad,), F32),
            pltpu.VMEM((ngd * GROUP,), I32),
        ],
    )


def _make_agg_kernel(n, ng, nchunks):
    npad = n + 8
    rpt = _round_up(-(-n // NS), 8)
    last = n - rpt * (NS - 1)
    assert last > 0
    half = nchunks // 2

    def body(*refs):
        vs = refs[:nchunks]
        edges_hbm = refs[nchunks]
        gs = refs[nchunks + 1:2 * nchunks + 1]
        (accum, packed_v, src_u, dst_u, rows0, rows1,
         sem_g0, sem_g1, sem_s0, sem_s1) = refs[2 * nchunks + 1:]
        rows = (rows0, rows1)
        sem_g = (sem_g0, sem_g1)
        sem_s = (sem_s0, sem_s1)
        cid = lax.axis_index("c")
        sid = lax.axis_index("s")
        pltpu.sync_copy(edges_hbm.at[sid], packed_v)

        def unpack(g, slot):
            for i in range(GROUP // 16):
                w = packed_v[pl.ds(g * GROUP + i * 16, 16)]
                src_u[slot, pl.ds(i * 16, 16)] = w & 0xFFFF
                dst_u[slot, pl.ds(i * 16, 16)] = lax.shift_right_logical(w, 16)

        def one_pass(v_ref, g_ref):
            off = pl.multiple_of(sid * rpt, 8)

            @pl.when(sid < NS - 1)
            def _():
                pltpu.sync_copy(v_ref.at[pl.ds(off, rpt)],
                                accum.at[pl.ds(off, rpt)])

            @pl.when(sid == NS - 1)
            def _():
                pltpu.sync_copy(v_ref.at[pl.ds((NS - 1) * rpt, last)],
                                accum.at[pl.ds((NS - 1) * rpt, last)])

            plsc.subcore_barrier()

            def gather(g, k):
                return pltpu.async_copy(v_ref.at[src_u.at[k % 4]],
                                        rows[k % 2], sem_g[k % 2])

            def wait_gather(g, k):
                pltpu.make_async_copy(v_ref.at[src_u.at[k % 4]],
                                      rows[k % 2], sem_g[k % 2]).wait()

            def scatter(g, k):
                return pltpu.async_copy(rows[k % 2],
                                        accum.at[dst_u.at[k % 4]],
                                        sem_s[k % 2], add=True)

            def wait_scatter(g, k):
                pltpu.make_async_copy(rows[k % 2],
                                      accum.at[dst_u.at[k % 4]],
                                      sem_s[k % 2]).wait()

            unpack(0, 0)
            unpack(1, 1)
            gather(0, 0)

            def quad(q, c):
                for k in range(4):
                    g = 4 * q + k
                    wait_gather(g, k)

                    @pl.when(g > 0)
                    def _():
                        wait_scatter(g - 1, k - 1)

                    scatter(g, k)

                    @pl.when(g + 2 < ng)
                    def _():
                        unpack(g + 2, (k + 2) % 4)

                    @pl.when(g + 1 < ng)
                    def _():
                        gather(g + 1, k + 1)
                return c
            lax.fori_loop(0, ng // 4, quad, 0)
            wait_scatter(ng - 1, (ng - 1) % 4)

            plsc.subcore_barrier()

            @pl.when(sid < NS - 1)
            def _():
                pltpu.sync_copy(accum.at[pl.ds(off, rpt)],
                                g_ref.at[pl.ds(off, rpt)])

            @pl.when(sid == NS - 1)
            def _():
                pltpu.sync_copy(accum.at[pl.ds((NS - 1) * rpt, last)],
                                g_ref.at[pl.ds((NS - 1) * rpt, last)])

        for p in range(half):
            @pl.when(cid == 0)
            def _(p=p):
                one_pass(vs[p], gs[p])

            @pl.when(cid == 1)
            def _(p=p):
                one_pass(vs[p + half], gs[p + half])

    return pl.kernel(
        body,
        out_type=tuple(jax.ShapeDtypeStruct((n, LANES), F32)
                       for _ in range(nchunks)),
        mesh=_sc_mesh(),
        scratch_types=[
            pltpu.VMEM_SHARED((npad, LANES), F32),
            pltpu.VMEM((ng * GROUP,), I32),
            pltpu.VMEM((4, GROUP), I32),
            pltpu.VMEM((4, GROUP), I32),
            pltpu.VMEM((GROUP, LANES), F32),
            pltpu.VMEM((GROUP, LANES), F32),
            pltpu.SemaphoreType.DMA,
            pltpu.SemaphoreType.DMA,
            pltpu.SemaphoreType.DMA,
            pltpu.SemaphoreType.DMA,
        ],
    )


def _prep(x, degp, n, bn):
    grid = -(-n // bn)
    nw, npad = degp.shape
    d_in = x.shape[1]

    def body(x_ref, dp_ref, srep_ref, x0_ref, x1_ref):
        d = jnp.sum(dp_ref[...], axis=0)[:, None] + 1.0
        s = lax.rsqrt(d)
        srep_ref[...] = jnp.broadcast_to(s, (bn, LANES))
        xb = x_ref[...]
        x0_ref[...] = xb[:, :LANES] * s
        x1_ref[...] = xb[:, LANES:] * s

    return pl.pallas_call(
        body,
        grid=(grid,),
        in_specs=[
            pl.BlockSpec((bn, d_in), lambda i: (i, 0)),
            pl.BlockSpec((nw, bn), lambda i: (0, i)),
        ],
        out_specs=[pl.BlockSpec((bn, LANES), lambda i: (i, 0))] * 3,
        out_shape=[jax.ShapeDtypeStruct((n, LANES), F32)] * 3,
    )(x, degp)


def _m1(g0, g1, srep, W1, b1, n, bn):
    grid = -(-n // bn)
    d_in, d_h = W1.shape

    def body(g0_ref, g1_ref, s_ref, w_ref, b_ref, *outs):
        scol = s_ref[...][:, 0:1]
        a = jnp.concatenate([g0_ref[...], g1_ref[...]], axis=1) * scol
        u = lax.dot_general(a, w_ref[...], (((1,), (0,)), ((), ())),
                            precision=HIGHEST, preferred_element_type=F32)
        h = jnp.maximum(u + b_ref[...], 0.0) * scol
        for c, o in enumerate(outs):
            o[...] = h[:, c * LANES:(c + 1) * LANES]

    nchunks = d_h // LANES
    return pl.pallas_call(
        body,
        grid=(grid,),
        in_specs=[
            pl.BlockSpec((bn, LANES), lambda i: (i, 0)),
            pl.BlockSpec((bn, LANES), lambda i: (i, 0)),
            pl.BlockSpec((bn, LANES), lambda i: (i, 0)),
            pl.BlockSpec((d_in, d_h), lambda i: (0, 0)),
            pl.BlockSpec((1, d_h), lambda i: (0, 0)),
        ],
        out_specs=[pl.BlockSpec((bn, LANES), lambda i: (i, 0))] * nchunks,
        out_shape=[jax.ShapeDtypeStruct((n, LANES), F32)] * nchunks,
    )(g0, g1, srep, W1, b1)


def _m23(gs, srep, W2, b2, W3, n, bn):
    grid = -(-n // bn)
    d_h = W2.shape[0]
    d_emb = W3.shape[1]
    nin = len(gs)

    def body(*refs):
        g_refs = refs[:nin]
        s_ref, w2_ref, b2_ref, w3_ref = refs[nin:nin + 4]
        outs = refs[nin + 4:]
        scol = s_ref[...][:, 0:1]
        a = jnp.concatenate([g[...] for g in g_refs], axis=1) * scol
        u = lax.dot_general(a, w2_ref[...], (((1,), (0,)), ((), ())),
                            precision=HIGHEST, preferred_element_type=F32)
        h = jnp.maximum(u + b2_ref[...], 0.0)
        t = lax.dot_general(h, w3_ref[...], (((1,), (0,)), ((), ())),
                            precision=HIGHEST, preferred_element_type=F32)
        t = t * scol
        for c, o in enumerate(outs):
            o[...] = t[:, c * LANES:(c + 1) * LANES]

    nchunks = d_emb // LANES
    return pl.pallas_call(
        body,
        grid=(grid,),
        in_specs=(
            [pl.BlockSpec((bn, LANES), lambda i: (i, 0))] * (nin + 1)
            + [
                pl.BlockSpec((d_h, d_h), lambda i: (0, 0)),
                pl.BlockSpec((1, d_h), lambda i: (0, 0)),
                pl.BlockSpec((d_h, d_emb), lambda i: (0, 0)),
            ]
        ),
        out_specs=[pl.BlockSpec((bn, LANES), lambda i: (i, 0))] * nchunks,
        out_shape=[jax.ShapeDtypeStruct((n, LANES), F32)] * nchunks,
    )(*gs, srep, W2, b2, W3)


def _pool(g0, g1, srep, b3, batch3, Wc, bc, n, bn, n_graphs):
    grid = -(-n // bn)
    d_emb = b3.shape[1]
    n_cls = Wc.shape[1]

    def body(g0_ref, g1_ref, s_ref, b3_ref, bt_ref, wc_ref, bc_ref,
             emb_ref, log_ref, sums, cnts):
        i = pl.program_id(0)

        @pl.when(i == 0)
        def _():
            sums[...] = jnp.zeros_like(sums)
            cnts[...] = jnp.zeros_like(cnts)

        scol = s_ref[...][:, 0:1]
        v = (jnp.concatenate([g0_ref[...], g1_ref[...]], axis=1) * scol
             + b3_ref[...])
        li = lax.broadcasted_iota(I32, (bn, 1), 0)
        valid = (i * bn + li) < n
        v = jnp.where(valid, v, 0.0)
        b = bt_ref[...][0, 0, :]
        gi = lax.broadcasted_iota(I32, (n_graphs, bn), 0)
        m = (gi == b[None, :]).astype(F32)
        sums[...] += lax.dot_general(m, v, (((1,), (0,)), ((), ())),
                                     precision=HIGHEST,
                                     preferred_element_type=F32)
        cnts[...] += jnp.broadcast_to(
            jnp.sum(m, axis=1, keepdims=True), cnts.shape)

        @pl.when(i == grid - 1)
        def _():
            cnt = jnp.maximum(cnts[...][:, 0:1], 1.0)
            emb = sums[...] / cnt
            emb_ref[...] = emb
            log_ref[...] = lax.dot_general(
                emb, wc_ref[...], (((1,), (0,)), ((), ())),
                precision=HIGHEST, preferred_element_type=F32) + bc_ref[...]

    return pl.pallas_call(
        body,
        grid=(grid,),
        in_specs=[
            pl.BlockSpec((bn, LANES), lambda i: (i, 0)),
            pl.BlockSpec((bn, LANES), lambda i: (i, 0)),
            pl.BlockSpec((bn, LANES), lambda i: (i, 0)),
            pl.BlockSpec((1, d_emb), lambda i: (0, 0)),
            pl.BlockSpec((1, 1, bn), lambda i: (i, 0, 0)),
            pl.BlockSpec((d_emb, n_cls), lambda i: (0, 0)),
            pl.BlockSpec((1, n_cls), lambda i: (0, 0)),
        ],
        out_specs=[
            pl.BlockSpec((n_graphs, d_emb), lambda i: (0, 0)),
            pl.BlockSpec((n_graphs, n_cls), lambda i: (0, 0)),
        ],
        out_shape=[
            jax.ShapeDtypeStruct((n_graphs, d_emb), F32),
            jax.ShapeDtypeStruct((n_graphs, n_cls), F32),
        ],
        scratch_shapes=[
            pltpu.VMEM((n_graphs, d_emb), F32),
            pltpu.VMEM((n_graphs, LANES), F32),
        ],
    )(g0, g1, srep, b3, batch3, Wc, bc)


def kernel(x, edge_index, batch, W1, b1, W2, b2, W3, b3, Wc, bc):
    n, d_in = x.shape
    e = edge_index.shape[1]
    n_graphs = 64
    bn = 1024
    src = edge_index[0].astype(I32)
    dst = edge_index[1].astype(I32)

    ept = _round_up(-(-e // NS), 4 * GROUP)
    ng = ept // GROUP
    pad16 = NS * ept - e
    edges16 = jnp.concatenate(
        [src | (dst << 16), jnp.full((pad16,), n << 16, I32)]).reshape(NS, ept)

    nw = NC * NS
    eptd = _round_up(-(-e // nw), GROUP)
    ngd = eptd // GROUP
    pad32 = nw * eptd - e
    dstd = jnp.concatenate(
        [dst, jnp.full((pad32,), n, I32)]).reshape(nw, eptd)

    npad_deg = _round_up(n + 1, bn)
    degp = _make_deg_kernel(n, ngd, npad_deg)(dstd)
    srep, x0, x1 = _prep(x, degp, n, bn)

    g10, g11 = _make_agg_kernel(n, ng, 2)(x0, x1, edges16)
    hs = _m1(g10, g11, srep, W1, b1.reshape(1, -1), n, bn)
    g2 = _make_agg_kernel(n, ng, 4)(*hs, edges16)
    t0, t1 = _m23(g2, srep, W2, b2.reshape(1, -1), W3, n, bn)
    g30, g31 = _make_agg_kernel(n, ng, 2)(t0, t1, edges16)

    ngrid = -(-n // bn)
    batch_pad = jnp.concatenate(
        [batch.astype(I32), jnp.full((ngrid * bn - n,), n_graphs, I32)])
    batch3 = batch_pad.reshape(ngrid, 1, bn)
    emb, logits = _pool(g30, g31, srep, b3.reshape(1, -1), batch3,
                        Wc, bc.reshape(1, -1), n, bn, n_graphs)
    return emb, logits


# speedup vs baseline: 1.4418x; 1.3091x over previous
import jax
import jax.numpy as jnp
from jax import lax
from jax.experimental import pallas as pl
from jax.experimental.pallas import tpu as pltpu
from jax.experimental.pallas import tpu_sc as plsc

F32 = jnp.float32
I32 = jnp.int32
NC = 2
NS = 16
GROUP = 128
LANES = 128
HIGHEST = lax.Precision.HIGHEST


def _round_up(a, b):
    return (a + b - 1) // b * b


def _sc_mesh():
    return plsc.VectorSubcoreMesh(
        core_axis_name="c", subcore_axis_name="s",
        num_cores=NC, num_subcores=NS)


def _make_deg_kernel(n, ngd, npad):
    nw = NC * NS

    def body(dst_hbm, out_hbm, deg_v, dst_v):
        cid = lax.axis_index("c")
        sid = lax.axis_index("s")
        wid = cid * NS + sid
        pltpu.sync_copy(dst_hbm.at[wid], dst_v)

        def zstep(i, c):
            deg_v[pl.ds(i * 16, 16)] = jnp.zeros((16,), F32)
            return c
        lax.fori_loop(0, npad // 16, zstep, 0)

        ones = jnp.ones((16,), F32)

        def estep(g, c):
            for j in range(GROUP // 16):
                idx = dst_v[pl.ds(g * GROUP + j * 16, 16)]
                plsc.addupdate_scatter(deg_v, [idx], ones)
            return c
        lax.fori_loop(0, ngd, estep, 0)
        pltpu.sync_copy(deg_v, out_hbm.at[wid])

    return pl.kernel(
        body,
        out_type=jax.ShapeDtypeStruct((nw, npad), F32),
        mesh=_sc_mesh(),
        compiler_params=pltpu.CompilerParams(needs_layout_passes=False),
        scratch_types=[
            pltpu.VMEM((npad,), F32),
            pltpu.VMEM((ngd * GROUP,), I32),
        ],
    )


def _make_agg_kernel(n, ng, nchunks):
    npad = n + 8
    rpt = _round_up(-(-n // NS), 8)
    last = n - rpt * (NS - 1)
    assert last > 0
    half = nchunks // 2

    def body(*refs):
        vs = refs[:nchunks]
        src_hbm = refs[nchunks]
        dst_hbm = refs[nchunks + 1]
        gs = refs[nchunks + 2:2 * nchunks + 2]
        accum, src_v, dst_v, rows, sem = refs[2 * nchunks + 2:]
        cid = lax.axis_index("c")
        sid = lax.axis_index("s")
        pltpu.sync_copy(src_hbm.at[sid], src_v)
        pltpu.sync_copy(dst_hbm.at[sid], dst_v)

        def one_pass(v_ref, g_ref):
            off = pl.multiple_of(sid * rpt, 8)

            @pl.when(sid < NS - 1)
            def _():
                pltpu.sync_copy(v_ref.at[pl.ds(off, rpt)],
                                accum.at[pl.ds(off, rpt)])

            @pl.when(sid == NS - 1)
            def _():
                pltpu.sync_copy(v_ref.at[pl.ds((NS - 1) * rpt, last)],
                                accum.at[pl.ds((NS - 1) * rpt, last)])

            plsc.subcore_barrier()

            def step(g, c):
                pltpu.async_copy(v_ref.at[src_v.at[g]], rows, sem).wait()
                pltpu.sync_copy(rows, accum.at[dst_v.at[g]], add=True)
                return c
            lax.fori_loop(0, ng, step, 0)

            plsc.subcore_barrier()

            @pl.when(sid < NS - 1)
            def _():
                pltpu.sync_copy(accum.at[pl.ds(off, rpt)],
                                g_ref.at[pl.ds(off, rpt)])

            @pl.when(sid == NS - 1)
            def _():
                pltpu.sync_copy(accum.at[pl.ds((NS - 1) * rpt, last)],
                                g_ref.at[pl.ds((NS - 1) * rpt, last)])

        for p in range(half):
            @pl.when(cid == 0)
            def _(p=p):
                one_pass(vs[p], gs[p])

            @pl.when(cid == 1)
            def _(p=p):
                one_pass(vs[p + half], gs[p + half])

    return pl.kernel(
        body,
        out_type=tuple(jax.ShapeDtypeStruct((n, LANES), F32)
                       for _ in range(nchunks)),
        mesh=_sc_mesh(),
        scratch_types=[
            pltpu.VMEM_SHARED((npad, LANES), F32),
            pltpu.VMEM((ng, GROUP), I32),
            pltpu.VMEM((ng, GROUP), I32),
            pltpu.VMEM((GROUP, LANES), F32),
            pltpu.SemaphoreType.DMA,
        ],
    )


def _prep(x, degp, n, bn):
    grid = -(-n // bn)
    nw, npad = degp.shape
    d_in = x.shape[1]

    def body(x_ref, dp_ref, srep_ref, x0_ref, x1_ref):
        d = jnp.sum(dp_ref[...], axis=0)[:, None] + 1.0
        s = lax.rsqrt(d)
        srep_ref[...] = jnp.broadcast_to(s, (bn, LANES))
        xb = x_ref[...]
        x0_ref[...] = xb[:, :LANES] * s
        x1_ref[...] = xb[:, LANES:] * s

    return pl.pallas_call(
        body,
        grid=(grid,),
        in_specs=[
            pl.BlockSpec((bn, d_in), lambda i: (i, 0)),
            pl.BlockSpec((nw, bn), lambda i: (0, i)),
        ],
        out_specs=[pl.BlockSpec((bn, LANES), lambda i: (i, 0))] * 3,
        out_shape=[jax.ShapeDtypeStruct((n, LANES), F32)] * 3,
    )(x, degp)


def _m1(g0, g1, srep, W1, b1, n, bn):
    grid = -(-n // bn)
    d_in, d_h = W1.shape

    def body(g0_ref, g1_ref, s_ref, w_ref, b_ref, *outs):
        scol = s_ref[...][:, 0:1]
        a = jnp.concatenate([g0_ref[...], g1_ref[...]], axis=1) * scol
        u = lax.dot_general(a, w_ref[...], (((1,), (0,)), ((), ())),
                            precision=HIGHEST, preferred_element_type=F32)
        h = jnp.maximum(u + b_ref[...], 0.0) * scol
        for c, o in enumerate(outs):
            o[...] = h[:, c * LANES:(c + 1) * LANES]

    nchunks = d_h // LANES
    return pl.pallas_call(
        body,
        grid=(grid,),
        in_specs=[
            pl.BlockSpec((bn, LANES), lambda i: (i, 0)),
            pl.BlockSpec((bn, LANES), lambda i: (i, 0)),
            pl.BlockSpec((bn, LANES), lambda i: (i, 0)),
            pl.BlockSpec((d_in, d_h), lambda i: (0, 0)),
            pl.BlockSpec((1, d_h), lambda i: (0, 0)),
        ],
        out_specs=[pl.BlockSpec((bn, LANES), lambda i: (i, 0))] * nchunks,
        out_shape=[jax.ShapeDtypeStruct((n, LANES), F32)] * nchunks,
    )(g0, g1, srep, W1, b1)


def _m23(gs, srep, W2, b2, W3, n, bn):
    grid = -(-n // bn)
    d_h = W2.shape[0]
    d_emb = W3.shape[1]
    nin = len(gs)

    def body(*refs):
        g_refs = refs[:nin]
        s_ref, w2_ref, b2_ref, w3_ref = refs[nin:nin + 4]
        outs = refs[nin + 4:]
        scol = s_ref[...][:, 0:1]
        a = jnp.concatenate([g[...] for g in g_refs], axis=1) * scol
        u = lax.dot_general(a, w2_ref[...], (((1,), (0,)), ((), ())),
                            precision=HIGHEST, preferred_element_type=F32)
        h = jnp.maximum(u + b2_ref[...], 0.0)
        t = lax.dot_general(h, w3_ref[...], (((1,), (0,)), ((), ())),
                            precision=HIGHEST, preferred_element_type=F32)
        t = t * scol
        for c, o in enumerate(outs):
            o[...] = t[:, c * LANES:(c + 1) * LANES]

    nchunks = d_emb // LANES
    return pl.pallas_call(
        body,
        grid=(grid,),
        in_specs=(
            [pl.BlockSpec((bn, LANES), lambda i: (i, 0))] * (nin + 1)
            + [
                pl.BlockSpec((d_h, d_h), lambda i: (0, 0)),
                pl.BlockSpec((1, d_h), lambda i: (0, 0)),
                pl.BlockSpec((d_h, d_emb), lambda i: (0, 0)),
            ]
        ),
        out_specs=[pl.BlockSpec((bn, LANES), lambda i: (i, 0))] * nchunks,
        out_shape=[jax.ShapeDtypeStruct((n, LANES), F32)] * nchunks,
    )(*gs, srep, W2, b2, W3)


def _pool(g0, g1, srep, b3, batch3, Wc, bc, n, bn, n_graphs):
    grid = -(-n // bn)
    d_emb = b3.shape[1]
    n_cls = Wc.shape[1]

    def body(g0_ref, g1_ref, s_ref, b3_ref, bt_ref, wc_ref, bc_ref,
             emb_ref, log_ref, sums, cnts):
        i = pl.program_id(0)

        @pl.when(i == 0)
        def _():
            sums[...] = jnp.zeros_like(sums)
            cnts[...] = jnp.zeros_like(cnts)

        scol = s_ref[...][:, 0:1]
        v = (jnp.concatenate([g0_ref[...], g1_ref[...]], axis=1) * scol
             + b3_ref[...])
        li = lax.broadcasted_iota(I32, (bn, 1), 0)
        valid = (i * bn + li) < n
        v = jnp.where(valid, v, 0.0)
        b = bt_ref[...][0, 0, :]
        gi = lax.broadcasted_iota(I32, (n_graphs, bn), 0)
        m = (gi == b[None, :]).astype(F32)
        sums[...] += lax.dot_general(m, v, (((1,), (0,)), ((), ())),
                                     precision=HIGHEST,
                                     preferred_element_type=F32)
        cnts[...] += jnp.broadcast_to(
            jnp.sum(m, axis=1, keepdims=True), cnts.shape)

        @pl.when(i == grid - 1)
        def _():
            cnt = jnp.maximum(cnts[...][:, 0:1], 1.0)
            emb = sums[...] / cnt
            emb_ref[...] = emb
            log_ref[...] = lax.dot_general(
                emb, wc_ref[...], (((1,), (0,)), ((), ())),
                precision=HIGHEST, preferred_element_type=F32) + bc_ref[...]

    return pl.pallas_call(
        body,
        grid=(grid,),
        in_specs=[
            pl.BlockSpec((bn, LANES), lambda i: (i, 0)),
            pl.BlockSpec((bn, LANES), lambda i: (i, 0)),
            pl.BlockSpec((bn, LANES), lambda i: (i, 0)),
            pl.BlockSpec((1, d_emb), lambda i: (0, 0)),
            pl.BlockSpec((1, 1, bn), lambda i: (i, 0, 0)),
            pl.BlockSpec((d_emb, n_cls), lambda i: (0, 0)),
            pl.BlockSpec((1, n_cls), lambda i: (0, 0)),
        ],
        out_specs=[
            pl.BlockSpec((n_graphs, d_emb), lambda i: (0, 0)),
            pl.BlockSpec((n_graphs, n_cls), lambda i: (0, 0)),
        ],
        out_shape=[
            jax.ShapeDtypeStruct((n_graphs, d_emb), F32),
            jax.ShapeDtypeStruct((n_graphs, n_cls), F32),
        ],
        scratch_shapes=[
            pltpu.VMEM((n_graphs, d_emb), F32),
            pltpu.VMEM((n_graphs, LANES), F32),
        ],
    )(g0, g1, srep, b3, batch3, Wc, bc)


def kernel(x, edge_index, batch, W1, b1, W2, b2, W3, b3, Wc, bc):
    n, d_in = x.shape
    e = edge_index.shape[1]
    n_graphs = 64
    bn = 1024
    src = edge_index[0].astype(I32)
    dst = edge_index[1].astype(I32)

    ept = _round_up(-(-e // NS), GROUP)
    ng = ept // GROUP
    pad16 = NS * ept - e
    src16 = jnp.concatenate(
        [src, jnp.zeros((pad16,), I32)]).reshape(NS, ng, GROUP)
    dst16 = jnp.concatenate(
        [dst, jnp.full((pad16,), n, I32)]).reshape(NS, ng, GROUP)

    nw = NC * NS
    eptd = _round_up(-(-e // nw), GROUP)
    ngd = eptd // GROUP
    pad32 = nw * eptd - e
    dstd = jnp.concatenate(
        [dst, jnp.full((pad32,), n, I32)]).reshape(nw, eptd)

    npad_deg = _round_up(n + 1, bn)
    degp = _make_deg_kernel(n, ngd, npad_deg)(dstd)
    srep, x0, x1 = _prep(x, degp, n, bn)

    g10, g11 = _make_agg_kernel(n, ng, 2)(x0, x1, src16, dst16)
    hs = _m1(g10, g11, srep, W1, b1.reshape(1, -1), n, bn)
    g2 = _make_agg_kernel(n, ng, 4)(*hs, src16, dst16)
    t0, t1 = _m23(g2, srep, W2, b2.reshape(1, -1), W3, n, bn)
    g30, g31 = _make_agg_kernel(n, ng, 2)(t0, t1, src16, dst16)

    ngrid = -(-n // bn)
    batch_pad = jnp.concatenate(
        [batch.astype(I32), jnp.full((ngrid * bn - n,), n_graphs, I32)])
    batch3 = batch_pad.reshape(ngrid, 1, bn)
    emb, logits = _pool(g30, g31, srep, b3.reshape(1, -1), batch3,
                        Wc, bc.reshape(1, -1), n, bn, n_graphs)
    return emb, logits
